# baseline (speedup 1.0000x reference)
import functools

import jax
import jax.numpy as jnp
from jax import lax
from jax.experimental import pallas as pl
from jax.experimental.pallas import tpu as pltpu
from jax.experimental.pallas import tpu_sc as plsc

NC = 2
NS = 16
NW = NC * NS

BLK = 128
DP = 128


@functools.partial(jax.jit, static_argnums=(2, 3, 4))
def _emb_call(xtp, w2, b, h, d):
    jpw = b // BLK // NW
    nu = h * jpw

    mesh = plsc.VectorSubcoreMesh(core_axis_name="c", subcore_axis_name="s")

    @functools.partial(
        pl.kernel,
        mesh=mesh,
        out_type=jax.ShapeDtypeStruct((h, d, b), jnp.float32),
        compiler_params=pltpu.CompilerParams(needs_layout_passes=False),
        scratch_types=[
            pltpu.VMEM((BLK,), jnp.int32),
            pltpu.VMEM((BLK,), jnp.int32),
            pltpu.VMEM((BLK,), jnp.int32),
            pltpu.VMEM((BLK,), jnp.int32),
            pltpu.VMEM((BLK, DP), jnp.float32),
            pltpu.VMEM((BLK, DP), jnp.float32),
            pltpu.VMEM((d, BLK), jnp.float32),
            pltpu.VMEM((d, BLK), jnp.float32),
            pltpu.SemaphoreType.DMA,
            pltpu.SemaphoreType.DMA,
            pltpu.SemaphoreType.DMA,
            pltpu.SemaphoreType.DMA,
        ],
    )
    def emb(xtp_hbm, w2_hbm, out_hbm,
            i0, i1, p0, p1, ga0, ga1, t0, t1, g0, g1, o0, o1):
        wid = lax.axis_index("s") * NC + lax.axis_index("c")
        j0 = wid * jpw
        idx = (i0, i1)
        pidx = (p0, p1)
        gbuf = (ga0, ga1)
        tbuf = (t0, t1)
        gsem = (g0, g1)
        osem = (o0, o1)
        nq = BLK // 16
        rowq = [lax.iota(jnp.int32, 16) + 16 * q for q in range(nq)]

        def unit_pos(u):
            hh = u // jpw
            b0 = (j0 + u % jpw) * BLK
            return hh, b0

        def load_and_fire(u, bf):
            hh, b0 = unit_pos(u)
            pltpu.sync_copy(xtp_hbm.at[pl.ds(hh * b + b0, BLK)], idx[bf])
            for q in range(nq):
                v = idx[bf][pl.ds(16 * q, 16)]
                pidx[bf][pl.ds(16 * q, 16)] = v >> 1
            pltpu.async_copy(w2_hbm.at[pidx[bf]], gbuf[bf], gsem[bf])

        def wait_gather(bf):
            pltpu.make_async_copy(
                w2_hbm.at[pl.ds(0, BLK)], gbuf[bf], gsem[bf]
            ).wait()

        def transpose(bf):
            colq = []
            for q in range(nq):
                v = idx[bf][pl.ds(16 * q, 16)]
                colq.append((v & 1) * d)
            for c in range(d):
                for q in range(nq):
                    vec = plsc.load_gather(gbuf[bf], [rowq[q], colq[q] + c])
                    tbuf[bf][c, pl.ds(16 * q, 16)] = vec

        def fire_out(u, bf):
            hh, b0 = unit_pos(u)
            pltpu.async_copy(
                tbuf[bf], out_hbm.at[hh, :, pl.ds(b0, BLK)], osem[bf]
            )

        def wait_out(bf):
            pltpu.make_async_copy(
                tbuf[bf], out_hbm.at[0, :, pl.ds(0, BLK)], osem[bf]
            ).wait()

        load_and_fire(0, 0)
        load_and_fire(1, 1)
        wait_gather(0)
        transpose(0)
        fire_out(0, 0)

        def body(k, carry):
            u0 = 2 + 2 * k
            wait_out(0)
            load_and_fire(u0, 0)
            wait_gather(1)
            transpose(1)
            fire_out(u0 - 1, 1)
            wait_out(1)
            load_and_fire(u0 + 1, 1)
            wait_gather(0)
            transpose(0)
            fire_out(u0, 0)
            return carry

        lax.fori_loop(0, (nu - 2) // 2, body, 0)

        wait_out(0)
        wait_gather(1)
        transpose(1)
        fire_out(nu - 1, 1)
        wait_out(1)

    return emb(xtp, w2)


def kernel(x, weight):
    b, h = x.shape
    v, d = weight.shape
    xtp = x.T.reshape(h * b)
    w2 = weight.reshape(v // 2, 2 * d)
    out3 = _emb_call(xtp, w2, b, h, d)
    return out3.transpose(2, 0, 1)


# speedup vs baseline: 1.6289x; 1.6289x over previous
import functools

import jax
import jax.numpy as jnp
from jax import lax
from jax.experimental import pallas as pl
from jax.experimental.pallas import tpu as pltpu
from jax.experimental.pallas import tpu_sc as plsc

NC = 2
NS = 16
NW = NC * NS

IDX_VEC = 128
CH = 640
G = CH // IDX_VEC


@functools.partial(jax.jit, static_argnums=(2, 3))
def _emb_call(xf, weight, n, d):
    n_per_w = n // NW
    nch = n_per_w // CH

    mesh = plsc.VectorSubcoreMesh(core_axis_name="c", subcore_axis_name="s")

    @functools.partial(
        pl.kernel,
        mesh=mesh,
        out_type=jax.ShapeDtypeStruct((n, d), jnp.float32),
        compiler_params=pltpu.CompilerParams(use_tc_tiling_on_sc=False),
        scratch_types=[
            pltpu.VMEM((2, G, IDX_VEC), jnp.int32),
            pltpu.VMEM((2, CH, d), jnp.float32),
            pltpu.SemaphoreType.DMA,
            pltpu.SemaphoreType.DMA,
            pltpu.SemaphoreType.DMA,
            pltpu.SemaphoreType.DMA,
            pltpu.SemaphoreType.DMA,
            pltpu.SemaphoreType.DMA,
        ],
    )
    def emb(idx_hbm, tbl_hbm, out_hbm, idx_v, rows_v, g0, g1, o0, o1, s0, s1):
        wid = lax.axis_index("s") * NC + lax.axis_index("c")
        row0 = wid * (n_per_w // IDX_VEC)
        obase = wid * n_per_w
        gsem = (g0, g1)
        osem = (o0, o1)
        isem = (s0, s1)

        def fire_idx(c, b):
            pltpu.async_copy(
                idx_hbm.at[pl.ds(row0 + c * G, G)], idx_v.at[b], isem[b]
            )

        def wait_idx(b):
            pltpu.make_async_copy(
                idx_hbm.at[pl.ds(0, G)], idx_v.at[b], isem[b]
            ).wait()

        def fire_gathers(c, b):
            for j in range(G):
                pltpu.async_copy(
                    tbl_hbm.at[idx_v.at[b, j]],
                    rows_v.at[b, pl.ds(j * IDX_VEC, IDX_VEC)],
                    gsem[b],
                )

        def wait_gathers(b):
            pltpu.make_async_copy(
                tbl_hbm.at[pl.ds(0, CH)], rows_v.at[b], gsem[b]
            ).wait()

        def fire_out(c, b):
            pltpu.async_copy(
                rows_v.at[b], out_hbm.at[pl.ds(obase + c * CH, CH)], osem[b]
            )

        def wait_out(b):
            pltpu.make_async_copy(
                rows_v.at[b], out_hbm.at[pl.ds(0, CH)], osem[b]
            ).wait()

        fire_idx(0, 0)
        wait_idx(0)
        fire_gathers(0, 0)
        fire_idx(1, 1)

        wait_idx(1)
        fire_gathers(1, 1)
        wait_gathers(0)
        fire_idx(2, 0)
        fire_out(0, 0)

        wait_out(0)
        wait_idx(0)
        fire_gathers(2, 0)
        wait_gathers(1)
        fire_idx(3, 1)
        fire_out(1, 1)

        def step(c, b):
            wait_out(b)
            wait_idx(b)
            fire_gathers(c, b)
            wait_gathers(1 - b)
            fire_idx(c + 1, 1 - b)
            fire_out(c - 1, 1 - b)

        def body(k, carry):
            c0 = 3 + 2 * k
            step(c0, 1)
            step(c0 + 1, 0)
            return carry

        lax.fori_loop(0, (nch - 4) // 2, body, 0)

        wait_out(1)
        wait_idx(1)
        fire_gathers(nch - 1, 1)
        wait_gathers(0)
        fire_out(nch - 2, 0)

        wait_gathers(1)
        fire_out(nch - 1, 1)
        wait_out(0)
        wait_out(1)

    return emb(xf, weight)


def kernel(x, weight):
    b, h = x.shape
    v, d = weight.shape
    n = b * h
    xf = x.reshape(n // IDX_VEC, IDX_VEC)
    out = _emb_call(xf, weight, n, d)
    return out.reshape(b, h, d)
